# baseline (speedup 1.0000x reference)
import jax
import jax.numpy as jnp
from jax import lax
from jax.experimental import pallas as pl

DIM = 1024
NE = 16
N = 4096
M = 256
S = 512
ITERS = 8
EPS = 1e-6


def _mm_kernel(x_ref, w_ref, o_ref):
    o_ref[...] = jnp.dot(x_ref[...].astype(jnp.bfloat16),
                         w_ref[...].astype(jnp.bfloat16),
                         preferred_element_type=jnp.float32)


def _fold_axis(a, axis, mode):
    while a.shape[axis] > 1:
        n = a.shape[axis]
        if mode == 'fold':
            lo = lax.slice_in_dim(a, 0, n // 2, axis=axis)
            hi = lax.slice_in_dim(a, n // 2, n, axis=axis)
            a = lo + hi
        else:
            sh = list(a.shape)
            sh[axis:axis + 1] = [n // 2, 2]
            r = a.reshape(sh)
            a = lax.index_in_dim(r, 0, axis + 1, keepdims=False) + \
                lax.index_in_dim(r, 1, axis + 1, keepdims=False)
    return a


def _sum_tokens(e):
    a = e.reshape(64, 8, 8, NE)
    a = _fold_axis(a, 0, 'adj')[0]
    a = _fold_axis(a, 0, 'fold')[0]
    a = _fold_axis(a, 0, 'fold')[0]
    return a.reshape(1, NE)


def _sum_experts(e):
    s = e
    while s.shape[1] > 1:
        h = s.shape[1] // 2
        s = s[:, :h] + s[:, h:]
    return s


def _split3(v):
    v0 = v.astype(jnp.bfloat16)
    r = v - v0.astype(jnp.float32)
    v1 = r.astype(jnp.bfloat16)
    v2 = (r - v1.astype(jnp.float32)).astype(jnp.bfloat16)
    return v0, v1, v2


def _select_kernel(lg_ref, idx_ref, gate_ref):
    lg = lg_ref[0]
    t = jnp.log(jnp.maximum(lg, EPS))
    for _ in range(ITERS):
        m0 = jnp.max(t, axis=0, keepdims=True)
        m0 = jnp.where(jnp.isfinite(m0), m0, 0.0)
        t = t - (jnp.log(_sum_tokens(jnp.exp(t - m0))) + m0)
        m1 = jnp.max(t, axis=1, keepdims=True)
        m1 = jnp.where(jnp.isfinite(m1), m1, 0.0)
        t = t - (jnp.log(_sum_experts(jnp.exp(t - m1))) + m1)
    g = jnp.exp(t)

    keys = lax.bitcast_convert_type(g, jnp.int32)
    thr = jnp.zeros((1, NE), jnp.int32)
    for bit in range(30, -1, -1):
        cand = thr | (1 << bit)
        cnt = jnp.sum((keys >= cand).astype(jnp.int32), axis=0, keepdims=True)
        thr = jnp.where(cnt >= M, cand, thr)

    mask = keys >= thr
    c = mask.astype(jnp.int32)
    sft = 1
    while sft < N:
        c = c + jnp.concatenate(
            [jnp.zeros((sft, NE), jnp.int32), c[:-sft]], axis=0)
        sft *= 2
    p = c - 1

    iota_n_col = lax.broadcasted_iota(jnp.int32, (N, 1), 0).astype(jnp.float32)
    iota_s_row = lax.broadcasted_iota(jnp.int32, (1, S), 1)
    iota_s_col = lax.broadcasted_iota(jnp.int32, (S, 1), 0)
    iota_m_row = lax.broadcasted_iota(jnp.int32, (1, M), 1)
    i0 = iota_n_col.astype(jnp.bfloat16)
    i1 = (iota_n_col - i0.astype(jnp.float32)).astype(jnp.bfloat16)

    for e in range(NE):
        pe = p[:, e:e + 1]
        me = mask[:, e:e + 1]
        v0, v1, v2 = _split3(g[:, e:e + 1])
        pay = jnp.concatenate([v0, v1, v2, i0, i1], axis=1)
        E = jnp.where((iota_s_row == pe) & me, 1.0, 0.0).astype(jnp.bfloat16)
        comp = lax.dot_general(E, pay, (((0,), (0,)), ((), ())),
                               preferred_element_type=jnp.float32)
        comp2 = lax.dot_general(pay, E, (((0,), (0,)), ((), ())),
                                preferred_element_type=jnp.float32)
        cv_c = (comp[:, 0:1] + comp[:, 1:2]) + comp[:, 2:3]
        cv_r = (comp2[0:1, :] + comp2[1:2, :]) + comp2[2:3, :]
        cnt_e = pe[N - 1:N, 0:1] + 1
        valid_c = iota_s_col < cnt_e
        valid_r = iota_s_row < cnt_e
        beats = (cv_r > cv_c) | ((cv_r == cv_c) & (iota_s_row < iota_s_col))
        A = valid_r & valid_c & beats
        r = jnp.sum(A.astype(jnp.int32), axis=1, keepdims=True)
        F = ((r == iota_m_row) & valid_c).astype(jnp.bfloat16)
        out5 = lax.dot_general(F, comp.astype(jnp.bfloat16),
                               (((0,), (0,)), ((), ())),
                               preferred_element_type=jnp.float32)
        val_o = (out5[:, 0:1] + out5[:, 1:2]) + out5[:, 2:3]
        idx_o = out5[:, 3:4] + out5[:, 4:5]
        idx_ref[0, :, e:e + 1] = idx_o.astype(jnp.int32)
        gate_ref[0, :, e:e + 1] = (val_o > 0.5).astype(jnp.float32)


def kernel(x, to_gate_weight):
    b, n, d = x.shape
    w = to_gate_weight[0]
    x2 = x.reshape(b * n, d)
    nchunk = 2048
    logits = pl.pallas_call(
        _mm_kernel,
        grid=(b * n // nchunk,),
        in_specs=[
            pl.BlockSpec((nchunk, d), lambda i: (i, 0)),
            pl.BlockSpec((d, NE), lambda i: (0, 0)),
        ],
        out_specs=pl.BlockSpec((nchunk, NE), lambda i: (i, 0)),
        out_shape=jax.ShapeDtypeStruct((b * n, NE), jnp.float32),
    )(x2, w).reshape(b, n, NE)

    idx, gate = pl.pallas_call(
        _select_kernel,
        grid=(b,),
        in_specs=[pl.BlockSpec((1, n, NE), lambda i: (i, 0, 0))],
        out_specs=[
            pl.BlockSpec((1, M, NE), lambda i: (i, 0, 0)),
            pl.BlockSpec((1, M, NE), lambda i: (i, 0, 0)),
        ],
        out_shape=[
            jax.ShapeDtypeStruct((b, M, NE), jnp.int32),
            jax.ShapeDtypeStruct((b, M, NE), jnp.float32),
        ],
    )(logits)
    return idx, gate


# speedup vs baseline: 3.9152x; 3.9152x over previous
import jax
import jax.numpy as jnp
from jax import lax
from jax.experimental import pallas as pl

DIM = 1024
NE = 16
N = 4096
M = 256
S = 320
NH = S // 64
ITERS = 8
EPS = 1e-6


def _mm_kernel(x_ref, w_ref, o_ref):
    o_ref[...] = jnp.dot(x_ref[...].astype(jnp.bfloat16),
                         w_ref[...].astype(jnp.bfloat16),
                         preferred_element_type=jnp.float32)


def _sum_tokens_t(e):
    a = e
    s = 64
    while s < N:
        a = a + jnp.concatenate([a[:, s:], a[:, :s]], axis=1)
        s *= 2
    a = a[:, :64]
    while a.shape[1] > 1:
        h = a.shape[1] // 2
        a = a[:, :h] + a[:, h:]
    return a


def _sum_experts_t(e):
    s = e
    while s.shape[0] > 1:
        h = s.shape[0] // 2
        s = s[:h] + s[h:]
    return s


def _split3(v):
    v0 = v.astype(jnp.bfloat16)
    r = v - v0.astype(jnp.float32)
    v1 = r.astype(jnp.bfloat16)
    v2 = (r - v1.astype(jnp.float32)).astype(jnp.bfloat16)
    return v0, v1, v2


def _select_kernel(lg_ref, idx_ref, gate_ref):
    lg = jnp.transpose(lg_ref[0])
    t = jnp.log(jnp.maximum(lg, EPS))
    for _ in range(ITERS):
        m0 = jnp.max(t, axis=1, keepdims=True)
        m0 = jnp.where(jnp.isfinite(m0), m0, 0.0)
        t = t - (jnp.log(_sum_tokens_t(jnp.exp(t - m0))) + m0)
        m1 = jnp.max(t, axis=0, keepdims=True)
        m1 = jnp.where(jnp.isfinite(m1), m1, 0.0)
        t = t - (jnp.log(_sum_experts_t(jnp.exp(t - m1))) + m1)
    g = jnp.exp(t)

    keys = lax.bitcast_convert_type(g, jnp.int32)
    thr = jnp.zeros((NE, 1), jnp.int32)
    for bit in range(30, -1, -1):
        cand = thr | (1 << bit)
        cnt = jnp.sum((keys >= cand).astype(jnp.int32), axis=1, keepdims=True)
        thr = jnp.where(cnt >= M, cand, thr)

    mask = keys >= thr
    c = mask.astype(jnp.int32)
    sft = 1
    while sft < N:
        c = c + jnp.concatenate(
            [jnp.zeros((NE, sft), jnp.int32), c[:, :-sft]], axis=1)
        sft *= 2
    p = c - 1

    iota_n_row = lax.broadcasted_iota(jnp.int32, (1, N), 1).astype(jnp.float32)
    iota64_col = lax.broadcasted_iota(jnp.int32, (64, 1), 0)
    iota_s_row = lax.broadcasted_iota(jnp.int32, (1, S), 1)
    iota_s_col = lax.broadcasted_iota(jnp.int32, (S, 1), 0)
    iota_m_row = lax.broadcasted_iota(jnp.int32, (1, M), 1)
    i0 = iota_n_row.astype(jnp.bfloat16)
    i1 = (iota_n_row - i0.astype(jnp.float32)).astype(jnp.bfloat16)

    for e in range(NE):
        pe = p[e:e + 1, :]
        me = mask[e:e + 1, :]
        v0, v1, v2 = _split3(g[e:e + 1, :])
        pay5 = jnp.concatenate([v0, v1, v2, i0, i1], axis=0)
        hi = jnp.right_shift(pe, 6)
        lo = jnp.bitwise_and(pe, 63)
        vs = [jnp.where((hi == h) & me, pay5, jnp.bfloat16(0.0))
              for h in range(NH)]
        vpay = jnp.concatenate(vs, axis=0)
        e_lo = jnp.where(iota64_col == lo, 1.0, 0.0).astype(jnp.bfloat16)
        c3 = lax.dot_general(e_lo, vpay, (((1,), (1,)), ((), ())),
                             preferred_element_type=jnp.float32)
        comp = jnp.concatenate(
            [c3[:, 5 * h:5 * (h + 1)] for h in range(NH)], axis=0)
        cv_c = (comp[:, 0:1] + comp[:, 1:2]) + comp[:, 2:3]
        ci_c = comp[:, 3:4] + comp[:, 4:5]
        cv_r = jnp.transpose(cv_c)
        cnt_e = pe[0:1, N - 1:N] + 1
        valid_c = iota_s_col < cnt_e
        valid_r = iota_s_row < cnt_e
        beats = (cv_r > cv_c) | ((cv_r == cv_c) & (iota_s_row < iota_s_col))
        A = valid_r & valid_c & beats
        r = jnp.sum(A.astype(jnp.int32), axis=1, keepdims=True)
        F = ((r == iota_m_row) & valid_c).astype(jnp.bfloat16)
        v0c, v1c, v2c = _split3(cv_c)
        i0c = ci_c.astype(jnp.bfloat16)
        i1c = (ci_c - i0c.astype(jnp.float32)).astype(jnp.bfloat16)
        pay_sc = jnp.concatenate([v0c, v1c, v2c, i0c, i1c], axis=1)
        out5 = lax.dot_general(F, pay_sc, (((0,), (0,)), ((), ())),
                               preferred_element_type=jnp.float32)
        val_o = (out5[:, 0:1] + out5[:, 1:2]) + out5[:, 2:3]
        idx_o = out5[:, 3:4] + out5[:, 4:5]
        idx_ref[0, :, e:e + 1] = idx_o.astype(jnp.int32)
        gate_ref[0, :, e:e + 1] = (val_o > 0.5).astype(jnp.float32)


def kernel(x, to_gate_weight):
    b, n, d = x.shape
    w = to_gate_weight[0]
    x2 = x.reshape(b * n, d)
    nchunk = 2048
    logits = pl.pallas_call(
        _mm_kernel,
        grid=(b * n // nchunk,),
        in_specs=[
            pl.BlockSpec((nchunk, d), lambda i: (i, 0)),
            pl.BlockSpec((d, NE), lambda i: (0, 0)),
        ],
        out_specs=pl.BlockSpec((nchunk, NE), lambda i: (i, 0)),
        out_shape=jax.ShapeDtypeStruct((b * n, NE), jnp.float32),
    )(x2, w).reshape(b, n, NE)

    idx, gate = pl.pallas_call(
        _select_kernel,
        grid=(b,),
        in_specs=[pl.BlockSpec((1, n, NE), lambda i: (i, 0, 0))],
        out_specs=[
            pl.BlockSpec((1, M, NE), lambda i: (i, 0, 0)),
            pl.BlockSpec((1, M, NE), lambda i: (i, 0, 0)),
        ],
        out_shape=[
            jax.ShapeDtypeStruct((b, M, NE), jnp.int32),
            jax.ShapeDtypeStruct((b, M, NE), jnp.float32),
        ],
    )(logits)
    return idx, gate


# speedup vs baseline: 4.0387x; 1.0316x over previous
import jax
import jax.numpy as jnp
from jax import lax
from jax.experimental import pallas as pl

DIM = 1024
NE = 16
N = 4096
M = 256
S = 320
NH = S // 64
ITERS = 8
EPS = 1e-6


def _mm_kernel(x_ref, w_ref, o_ref):
    o_ref[...] = jnp.dot(x_ref[...].astype(jnp.bfloat16),
                         w_ref[...].astype(jnp.bfloat16),
                         preferred_element_type=jnp.float32)


def _sum_tokens_t(e):
    a = e
    s = 64
    while s < N:
        a = a + jnp.concatenate([a[:, s:], a[:, :s]], axis=1)
        s *= 2
    a = a[:, :64]
    while a.shape[1] > 1:
        h = a.shape[1] // 2
        a = a[:, :h] + a[:, h:]
    return a


def _sum_experts_t(e):
    s = e
    while s.shape[0] > 1:
        h = s.shape[0] // 2
        s = s[:h] + s[h:]
    return s


def _split3(v):
    v0 = v.astype(jnp.bfloat16)
    r = v - v0.astype(jnp.float32)
    v1 = r.astype(jnp.bfloat16)
    v2 = (r - v1.astype(jnp.float32)).astype(jnp.bfloat16)
    return v0, v1, v2


def _select_kernel(x_ref, w_ref, idx_ref, gate_ref):
    logits = jnp.dot(x_ref[0].astype(jnp.bfloat16),
                     w_ref[...].astype(jnp.bfloat16),
                     preferred_element_type=jnp.float32)
    lg = jnp.transpose(logits)
    t = jnp.log(jnp.maximum(lg, EPS))
    for _ in range(ITERS):
        m0 = jnp.max(t, axis=1, keepdims=True)
        m0 = jnp.where(jnp.isfinite(m0), m0, 0.0)
        t = t - (jnp.log(_sum_tokens_t(jnp.exp(t - m0))) + m0)
        m1 = jnp.max(t, axis=0, keepdims=True)
        m1 = jnp.where(jnp.isfinite(m1), m1, 0.0)
        t = t - (jnp.log(_sum_experts_t(jnp.exp(t - m1))) + m1)
    g = jnp.exp(t)

    keys = lax.bitcast_convert_type(g, jnp.int32)
    thr = jnp.zeros((NE, 1), jnp.int32)
    for bit in range(30, -1, -1):
        cand = thr | (1 << bit)
        cnt = jnp.sum((keys >= cand).astype(jnp.int32), axis=1, keepdims=True)
        thr = jnp.where(cnt >= M, cand, thr)

    mask = keys >= thr
    c = mask.astype(jnp.int32)
    sft = 1
    while sft < N:
        c = c + jnp.concatenate(
            [jnp.zeros((NE, sft), jnp.int32), c[:, :-sft]], axis=1)
        sft *= 2
    p = c - 1

    iota_n_row = lax.broadcasted_iota(jnp.int32, (1, N), 1).astype(jnp.float32)
    iota64_col = lax.broadcasted_iota(jnp.int32, (64, 1), 0)
    iota_s_row = lax.broadcasted_iota(jnp.int32, (1, S), 1)
    iota_s_col = lax.broadcasted_iota(jnp.int32, (S, 1), 0)
    iota_m_row = lax.broadcasted_iota(jnp.int32, (1, M), 1)
    i0 = iota_n_row.astype(jnp.bfloat16)
    i1 = (iota_n_row - i0.astype(jnp.float32)).astype(jnp.bfloat16)

    for e in range(NE):
        pe = p[e:e + 1, :]
        me = mask[e:e + 1, :]
        v0, v1, v2 = _split3(g[e:e + 1, :])
        pay5 = jnp.concatenate([v0, v1, v2, i0, i1], axis=0)
        hi = jnp.right_shift(pe, 6)
        lo = jnp.bitwise_and(pe, 63)
        vs = [jnp.where((hi == h) & me, pay5, jnp.bfloat16(0.0))
              for h in range(NH)]
        vpay = jnp.concatenate(vs, axis=0)
        e_lo = jnp.where(iota64_col == lo, 1.0, 0.0).astype(jnp.bfloat16)
        c3 = lax.dot_general(e_lo, vpay, (((1,), (1,)), ((), ())),
                             preferred_element_type=jnp.float32)
        comp = jnp.concatenate(
            [c3[:, 5 * h:5 * (h + 1)] for h in range(NH)], axis=0)
        cv_c = (comp[:, 0:1] + comp[:, 1:2]) + comp[:, 2:3]
        ci_c = comp[:, 3:4] + comp[:, 4:5]
        cv_r = jnp.transpose(cv_c)
        cnt_e = pe[0:1, N - 1:N] + 1
        valid_c = iota_s_col < cnt_e
        valid_r = iota_s_row < cnt_e
        beats = (cv_r > cv_c) | ((cv_r == cv_c) & (iota_s_row < iota_s_col))
        A = valid_r & valid_c & beats
        r = jnp.sum(A.astype(jnp.int32), axis=1, keepdims=True)
        F = ((r == iota_m_row) & valid_c).astype(jnp.bfloat16)
        v0c, v1c, v2c = _split3(cv_c)
        i0c = ci_c.astype(jnp.bfloat16)
        i1c = (ci_c - i0c.astype(jnp.float32)).astype(jnp.bfloat16)
        pay_sc = jnp.concatenate([v0c, v1c, v2c, i0c, i1c], axis=1)
        out5 = lax.dot_general(F, pay_sc, (((0,), (0,)), ((), ())),
                               preferred_element_type=jnp.float32)
        val_o = (out5[:, 0:1] + out5[:, 1:2]) + out5[:, 2:3]
        idx_o = out5[:, 3:4] + out5[:, 4:5]
        idx_ref[0, :, e:e + 1] = idx_o.astype(jnp.int32)
        gate_ref[0, :, e:e + 1] = (val_o > 0.5).astype(jnp.float32)


def kernel(x, to_gate_weight):
    b, n, d = x.shape
    w = to_gate_weight[0]
    idx, gate = pl.pallas_call(
        _select_kernel,
        grid=(b,),
        in_specs=[
            pl.BlockSpec((1, n, d), lambda i: (i, 0, 0)),
            pl.BlockSpec((d, NE), lambda i: (0, 0)),
        ],
        out_specs=[
            pl.BlockSpec((1, M, NE), lambda i: (i, 0, 0)),
            pl.BlockSpec((1, M, NE), lambda i: (i, 0, 0)),
        ],
        out_shape=[
            jax.ShapeDtypeStruct((b, M, NE), jnp.int32),
            jax.ShapeDtypeStruct((b, M, NE), jnp.float32),
        ],
    )(x, w)
    return idx, gate


# speedup vs baseline: 4.3243x; 1.0707x over previous
import jax
import jax.numpy as jnp
from jax import lax
from jax.experimental import pallas as pl

DIM = 1024
NE = 16
N = 4096
M = 256
S = 320
NH = S // 64
ITERS = 8
EPS = 1e-6


def _mm_kernel(x_ref, w_ref, o_ref):
    o_ref[...] = jnp.dot(x_ref[...].astype(jnp.bfloat16),
                         w_ref[...].astype(jnp.bfloat16),
                         preferred_element_type=jnp.float32)


def _sum_tokens_t(e):
    a = e
    s = 64
    while s < N:
        a = a + jnp.concatenate([a[:, s:], a[:, :s]], axis=1)
        s *= 2
    a = a[:, :64]
    while a.shape[1] > 1:
        h = a.shape[1] // 2
        a = a[:, :h] + a[:, h:]
    return a


def _sum_experts_t(e):
    s = e
    while s.shape[0] > 1:
        h = s.shape[0] // 2
        s = s[:h] + s[h:]
    return s


def _split3(v):
    v0 = v.astype(jnp.bfloat16)
    r = v - v0.astype(jnp.float32)
    v1 = r.astype(jnp.bfloat16)
    v2 = (r - v1.astype(jnp.float32)).astype(jnp.bfloat16)
    return v0, v1, v2


def _select_kernel(x_ref, wt_ref, idx_ref, gate_ref):
    lg = lax.dot_general(wt_ref[...].astype(jnp.bfloat16),
                         x_ref[0].astype(jnp.bfloat16),
                         (((1,), (1,)), ((), ())),
                         preferred_element_type=jnp.float32)
    t = jnp.log(jnp.maximum(lg, EPS))
    for _ in range(ITERS):
        m0 = jnp.max(t, axis=1, keepdims=True)
        m0 = jnp.where(jnp.isfinite(m0), m0, 0.0)
        t = t - (jnp.log(_sum_tokens_t(jnp.exp(t - m0))) + m0)
        m1 = jnp.max(t, axis=0, keepdims=True)
        m1 = jnp.where(jnp.isfinite(m1), m1, 0.0)
        t = t - (jnp.log(_sum_experts_t(jnp.exp(t - m1))) + m1)
    g = jnp.exp(t)

    keys = lax.bitcast_convert_type(g, jnp.int32)
    thr = jnp.zeros((NE, 1), jnp.int32)
    for bit in range(30, -1, -1):
        cand = thr | (1 << bit)
        cnt = jnp.sum((keys >= cand).astype(jnp.int32), axis=1, keepdims=True)
        thr = jnp.where(cnt >= M, cand, thr)

    mask = keys >= thr
    c = mask.astype(jnp.int32)
    sft = 1
    while sft < N:
        c = c + jnp.concatenate(
            [jnp.zeros((NE, sft), jnp.int32), c[:, :-sft]], axis=1)
        sft *= 2
    p = c - 1

    iota_n_row = lax.broadcasted_iota(jnp.int32, (1, N), 1).astype(jnp.float32)
    iota64_col = lax.broadcasted_iota(jnp.int32, (64, 1), 0)
    iota_s_row = lax.broadcasted_iota(jnp.int32, (1, S), 1)
    iota_s_col = lax.broadcasted_iota(jnp.int32, (S, 1), 0)
    iota_m_row = lax.broadcasted_iota(jnp.int32, (1, M), 1)
    i0 = iota_n_row.astype(jnp.bfloat16)
    i1 = (iota_n_row - i0.astype(jnp.float32)).astype(jnp.bfloat16)

    for e in range(NE):
        pe = p[e:e + 1, :]
        me = mask[e:e + 1, :]
        v0, v1, v2 = _split3(g[e:e + 1, :])
        pay5 = jnp.concatenate([v0, v1, v2, i0, i1], axis=0)
        hi = jnp.right_shift(pe, 6)
        lo = jnp.bitwise_and(pe, 63)
        vs = [jnp.where((hi == h) & me, pay5, jnp.bfloat16(0.0))
              for h in range(NH)]
        vpay = jnp.concatenate(vs, axis=0)
        e_lo = jnp.where(iota64_col == lo, 1.0, 0.0).astype(jnp.bfloat16)
        c3 = lax.dot_general(e_lo, vpay, (((1,), (1,)), ((), ())),
                             preferred_element_type=jnp.float32)
        comp = jnp.concatenate(
            [c3[:, 5 * h:5 * (h + 1)] for h in range(NH)], axis=0)
        cv_c = (comp[:, 0:1] + comp[:, 1:2]) + comp[:, 2:3]
        ci_c = comp[:, 3:4] + comp[:, 4:5]
        cv_r = jnp.transpose(cv_c)
        cnt_e = pe[0:1, N - 1:N] + 1
        valid_c = iota_s_col < cnt_e
        valid_r = iota_s_row < cnt_e
        beats = (cv_r > cv_c) | ((cv_r == cv_c) & (iota_s_row < iota_s_col))
        A = valid_r & valid_c & beats
        r = jnp.sum(A.astype(jnp.int32), axis=1, keepdims=True)
        F = ((r == iota_m_row) & valid_c).astype(jnp.bfloat16)
        v0c, v1c, v2c = _split3(cv_c)
        i0c = ci_c.astype(jnp.bfloat16)
        i1c = (ci_c - i0c.astype(jnp.float32)).astype(jnp.bfloat16)
        pay_sc = jnp.concatenate([v0c, v1c, v2c, i0c, i1c], axis=1)
        out5 = lax.dot_general(F, pay_sc, (((0,), (0,)), ((), ())),
                               preferred_element_type=jnp.float32)
        val_o = (out5[:, 0:1] + out5[:, 1:2]) + out5[:, 2:3]
        idx_o = out5[:, 3:4] + out5[:, 4:5]
        idx_ref[0, :, e:e + 1] = idx_o.astype(jnp.int32)
        gate_ref[0, :, e:e + 1] = (val_o > 0.5).astype(jnp.float32)


def kernel(x, to_gate_weight):
    b, n, d = x.shape
    wt = to_gate_weight[0].T
    idx, gate = pl.pallas_call(
        _select_kernel,
        grid=(b,),
        in_specs=[
            pl.BlockSpec((1, n, d), lambda i: (i, 0, 0)),
            pl.BlockSpec((NE, d), lambda i: (0, 0)),
        ],
        out_specs=[
            pl.BlockSpec((1, M, NE), lambda i: (i, 0, 0)),
            pl.BlockSpec((1, M, NE), lambda i: (i, 0, 0)),
        ],
        out_shape=[
            jax.ShapeDtypeStruct((b, M, NE), jnp.int32),
            jax.ShapeDtypeStruct((b, M, NE), jnp.float32),
        ],
    )(x, wt)
    return idx, gate
